# Initial kernel scaffold; baseline (speedup 1.0000x reference)
import jax
import jax.numpy as jnp
from jax.experimental import pallas as pl


def kernel(des, tweet, num_prop, cat_prop, new_feature, edge_index, edge_type, W_des, b_des, W_tweet, b_tweet, W_num, b_num, W_cat, b_cat, W_new, b_new, W_in, b_in, rgcn1_w, rgcn1_root, rgcn1_b, rgcn2_w, rgcn2_root, rgcn2_b, W_out1, b_out1, W_out2, b_out2):
    raise NotImplementedError("write your pallas kernel here")


# speedup vs baseline: 5.6812x; 5.6812x over previous
import functools

import jax
import jax.numpy as jnp
from jax import lax
from jax.experimental import pallas as pl
from jax.experimental.pallas import tpu as pltpu
from jax.experimental.pallas import tpu_sc as plsc

N = 10000
E = 320000
EMB = 128
NC = 2
NS = 16
NW = NC * NS
ECH = E // NS
CAP = ECH + 224
BK = 64
NB = CAP // BK
NPAD = 10112
RPT = NPAD // NS
TRASH = N
GK = 2
FP32 = jnp.float32


def _leaky(x):
    return jnp.where(x >= 0, x, 0.01 * x)


def _dot(a, b):
    return jnp.dot(a, b, preferred_element_type=FP32,
                   precision=lax.Precision.HIGHEST)


def _part_body(src_hbm, dst_hbm, et_hbm, psrc_hbm, pdst_hbm, pcnt_hbm,
               cnt_hbm, srcv, dstv, etv, psb, pdb, cntv, cntl):
    c = lax.axis_index("c")
    s = lax.axis_index("s")
    wid = s * NC + c
    base = s * ECH
    pltpu.sync_copy(src_hbm.at[pl.ds(base, ECH)], srcv)
    pltpu.sync_copy(dst_hbm.at[pl.ds(base, ECH)], dstv)
    pltpu.sync_copy(et_hbm.at[pl.ds(base, ECH)], etv)

    zeros16f = jnp.zeros((16,), FP32)

    def zbody(i, carry):
        cntl[pl.ds(i * 16, 16)] = zeros16f
        return carry

    lax.fori_loop(0, NPAD // 16, zbody, jnp.int32(0))

    ones16f = jnp.ones((16,), FP32)

    def body(i, n):
        sl = pl.ds(i * 16, 16)
        m = etv[sl] == c
        mv = m.astype(jnp.int32)
        pos = n + plsc.cumsum(mv) - 1
        plsc.store_scatter(psb, [pos], srcv[sl], mask=m)
        plsc.store_scatter(pdb, [pos], dstv[sl], mask=m)
        plsc.addupdate_scatter(cntl, [dstv[sl]], ones16f, mask=m)
        return n + jnp.sum(mv)

    n = lax.fori_loop(0, ECH // 16, body, jnp.int32(0))
    pltpu.sync_copy(cntl, cnt_hbm.at[wid])

    zeros16 = jnp.zeros((16,), jnp.int32)
    trash16 = jnp.full((16,), TRASH, jnp.int32)
    for t in range(8):
        psb[pl.ds(n + t * 16, 16)] = zeros16
        pdb[pl.ds(n + t * 16, 16)] = trash16

    cntv[...] = jnp.full((16,), n, jnp.int32)
    pltpu.sync_copy(psb, psrc_hbm.at[wid])
    pltpu.sync_copy(pdb, pdst_hbm.at[wid])
    pltpu.sync_copy(cntv, pcnt_hbm.at[wid])


_partition = pl.kernel(
    _part_body,
    out_type=[
        jax.ShapeDtypeStruct((NW, CAP), jnp.int32),
        jax.ShapeDtypeStruct((NW, CAP), jnp.int32),
        jax.ShapeDtypeStruct((NW, 16), jnp.int32),
        jax.ShapeDtypeStruct((NW, NPAD), FP32),
    ],
    mesh=plsc.VectorSubcoreMesh(core_axis_name="c", subcore_axis_name="s",
                                num_cores=NC, num_subcores=NS),
    scratch_types=[
        pltpu.VMEM((ECH,), jnp.int32),
        pltpu.VMEM((ECH,), jnp.int32),
        pltpu.VMEM((ECH,), jnp.int32),
        pltpu.VMEM((CAP,), jnp.int32),
        pltpu.VMEM((CAP,), jnp.int32),
        pltpu.VMEM((16,), jnp.int32),
        pltpu.VMEM((NPAD,), FP32),
    ],
    compiler_params=pltpu.CompilerParams(needs_layout_passes=False),
)


NZ = RPT // BK
TAIL = RPT - NZ * BK


def _agg_body(x_hbm, psrc_hbm, pdst_hbm, pcnt_hbm, zrow_hbm, sums_hbm,
              psb, pdb, pcb, rowbuf, idxb, sem, acc_sh):
    c = lax.axis_index("c")
    s = lax.axis_index("s")
    wid = s * NC + c
    row0 = s * RPT

    pltpu.sync_copy(pcnt_hbm.at[wid], pcb)
    n = jnp.max(pcb[...])
    nb = (n + BK - 1) // BK

    pltpu.sync_copy(zrow_hbm.at[pl.ds(0, BK)], rowbuf)
    last = row0 + RPT - 1
    for b in range(NZ + 1):
        for k in range(BK // 16):
            sl = pl.ds(k * 16, 16)
            idxb[sl] = jnp.minimum(
                lax.iota(jnp.int32, 16) + (row0 + b * BK + k * 16), last)
        pltpu.sync_copy(rowbuf, acc_sh.at[idxb])

    plsc.subcore_barrier()

    def step(j, carry):
        base = wid * CAP + j * BK
        pltpu.sync_copy(psrc_hbm.at[pl.ds(base, BK)], psb)
        pltpu.sync_copy(pdst_hbm.at[pl.ds(base, BK)], pdb)
        pltpu.async_copy(x_hbm.at[psb], rowbuf, sem).wait()
        pltpu.sync_copy(rowbuf, acc_sh.at[pdb], add=True)
        return carry

    lax.fori_loop(0, nb, step, jnp.int32(0))

    plsc.subcore_barrier()

    out0 = c * NPAD + row0
    for b in range(NZ + 1):
        w = BK if b < NZ else TAIL
        for k in range(BK // 16):
            sl = pl.ds(k * 16, 16)
            idxb[sl] = jnp.minimum(
                lax.iota(jnp.int32, 16) + (row0 + b * BK + k * 16), last)
        pltpu.async_copy(acc_sh.at[idxb], rowbuf, sem).wait()
        pltpu.sync_copy(rowbuf.at[pl.ds(0, w)],
                        sums_hbm.at[pl.ds(out0 + b * BK, w)])


_aggregate = pl.kernel(
    _agg_body,
    out_type=jax.ShapeDtypeStruct((NC * NPAD, 128), FP32),
    mesh=plsc.VectorSubcoreMesh(core_axis_name="c", subcore_axis_name="s",
                                num_cores=NC, num_subcores=NS),
    scratch_types=[
        pltpu.VMEM((BK,), jnp.int32),
        pltpu.VMEM((BK,), jnp.int32),
        pltpu.VMEM((16,), jnp.int32),
        pltpu.VMEM((BK, 128), FP32),
        pltpu.VMEM((BK,), jnp.int32),
        pltpu.SemaphoreType.DMA,
        pltpu.VMEM_SHARED((NPAD, 128), FP32),
    ],
    compiler_params=pltpu.CompilerParams(needs_layout_passes=False),
)


R = 400
GRID = N // R


def _t0_body(des_ref, tw_ref, sm_ref, wd_ref, wt_ref, ws_ref, bf_ref,
             win_ref, bin_ref, out_ref):
    h = (_dot(des_ref[...], wd_ref[...]) + _dot(tw_ref[...], wt_ref[...])
         + _dot(sm_ref[...], ws_ref[...]) + bf_ref[...])
    h = _leaky(h)
    out_ref[...] = _leaky(_dot(h, win_ref[...]) + bin_ref[...])


_t0 = pl.pallas_call(
    _t0_body,
    out_shape=jax.ShapeDtypeStruct((N, 128), FP32),
    grid=(GRID,),
    in_specs=[
        pl.BlockSpec((R, 768), lambda i: (i, 0)),
        pl.BlockSpec((R, 768), lambda i: (i, 0)),
        pl.BlockSpec((R, 24), lambda i: (i, 0)),
        pl.BlockSpec((768, 128), lambda i: (0, 0)),
        pl.BlockSpec((768, 128), lambda i: (0, 0)),
        pl.BlockSpec((24, 128), lambda i: (0, 0)),
        pl.BlockSpec((1, 128), lambda i: (0, 0)),
        pl.BlockSpec((128, 128), lambda i: (0, 0)),
        pl.BlockSpec((1, 128), lambda i: (0, 0)),
    ],
    out_specs=pl.BlockSpec((R, 128), lambda i: (i, 0)),
)


def _combine(s0, s1, c0, c1, x, w0, w1, wr, b):
    inv0 = 1.0 / jnp.maximum(jnp.sum(c0, axis=1, keepdims=True), 1.0)
    inv1 = 1.0 / jnp.maximum(jnp.sum(c1, axis=1, keepdims=True), 1.0)
    return (_dot(s0 * inv0, w0) + _dot(s1 * inv1, w1) + _dot(x, wr) + b)


def _t1_body(s0_ref, s1_ref, c0_ref, c1_ref, x_ref, w0_ref, w1_ref, wr_ref,
             b_ref, out_ref):
    out_ref[...] = _combine(s0_ref[0], s1_ref[0], c0_ref[0], c1_ref[0],
                            x_ref[...], w0_ref[...], w1_ref[...], wr_ref[...],
                            b_ref[...])


def _t1h_body(s0_ref, s1_ref, c0_ref, c1_ref, x_ref, w0_ref, w1_ref, wr_ref,
              b_ref, wo1_ref, bo1_ref, wo2_ref, bo2_ref, out_ref):
    y = _combine(s0_ref[0], s1_ref[0], c0_ref[0], c1_ref[0], x_ref[...],
                 w0_ref[...], w1_ref[...], wr_ref[...], b_ref[...])
    z = _leaky(_dot(y, wo1_ref[...]) + bo1_ref[...])
    out_ref[...] = _dot(z, wo2_ref[...]) + bo2_ref[...]


_COMBINE_SPECS = [
    pl.BlockSpec((1, R, 128), lambda i: (0, i, 0)),
    pl.BlockSpec((1, R, 128), lambda i: (1, i, 0)),
    pl.BlockSpec((1, R, 16), lambda i: (0, i, 0)),
    pl.BlockSpec((1, R, 16), lambda i: (1, i, 0)),
    pl.BlockSpec((R, 128), lambda i: (i, 0)),
    pl.BlockSpec((128, 128), lambda i: (0, 0)),
    pl.BlockSpec((128, 128), lambda i: (0, 0)),
    pl.BlockSpec((128, 128), lambda i: (0, 0)),
    pl.BlockSpec((1, 128), lambda i: (0, 0)),
]

_t1 = pl.pallas_call(
    _t1_body,
    out_shape=jax.ShapeDtypeStruct((N, 128), FP32),
    grid=(GRID,),
    in_specs=_COMBINE_SPECS,
    out_specs=pl.BlockSpec((R, 128), lambda i: (i, 0)),
)

_t1h = pl.pallas_call(
    _t1h_body,
    out_shape=jax.ShapeDtypeStruct((N, 128), FP32),
    grid=(GRID,),
    in_specs=_COMBINE_SPECS + [
        pl.BlockSpec((128, 128), lambda i: (0, 0)),
        pl.BlockSpec((1, 128), lambda i: (0, 0)),
        pl.BlockSpec((128, 128), lambda i: (0, 0)),
        pl.BlockSpec((1, 128), lambda i: (0, 0)),
    ],
    out_specs=pl.BlockSpec((R, 128), lambda i: (i, 0)),
)


def kernel(des, tweet, num_prop, cat_prop, new_feature, edge_index, edge_type,
           W_des, b_des, W_tweet, b_tweet, W_num, b_num, W_cat, b_cat, W_new,
           b_new, W_in, b_in, rgcn1_w, rgcn1_root, rgcn1_b, rgcn2_w,
           rgcn2_root, rgcn2_b, W_out1, b_out1, W_out2, b_out2):
    src = edge_index[0]
    dst = edge_index[1]

    small = jnp.concatenate(
        [num_prop, cat_prop, new_feature, jnp.zeros((N, 5), FP32)], axis=1)
    wd = jnp.zeros((768, 128), FP32).at[:, 0:25].set(W_des)
    wt = jnp.zeros((768, 128), FP32).at[:, 25:53].set(W_tweet)
    ws = (jnp.zeros((24, 128), FP32)
          .at[0:7, 53:78].set(W_num)
          .at[7:18, 78:103].set(W_cat)
          .at[18:19, 103:128].set(W_new))
    bf = jnp.concatenate([b_des, b_tweet, b_num, b_cat, b_new]).reshape(1, 128)

    x0 = _t0(des, tweet, small, wd, wt, ws, bf, W_in, b_in.reshape(1, 128))

    psrc, pdst, pcnt, cnt = _partition(src, dst, edge_type)
    psrc3 = psrc.reshape(NW * CAP)
    pdst3 = pdst.reshape(NW * CAP)
    cnts = cnt.reshape(NS, NC, NPAD).transpose(1, 2, 0)

    zrow = jnp.zeros((RPT, 128), FP32)

    sums1 = _aggregate(x0, psrc3, pdst3, pcnt, zrow).reshape(NC, NPAD, 128)

    x1 = _t1(sums1, sums1, cnts, cnts, x0, rgcn1_w[0], rgcn1_w[1],
             rgcn1_root, rgcn1_b.reshape(1, 128))

    sums2 = _aggregate(x1, psrc3, pdst3, pcnt, zrow).reshape(NC, NPAD, 128)

    wo2 = jnp.zeros((128, 128), FP32).at[:, 0:2].set(W_out2)
    bo2 = jnp.zeros((1, 128), FP32).at[0, 0:2].set(b_out2)

    out = _t1h(sums2, sums2, cnts, cnts, x1, rgcn2_w[0], rgcn2_w[1],
               rgcn2_root, rgcn2_b.reshape(1, 128), W_out1,
               b_out1.reshape(1, 128), wo2, bo2)
    return out[:, 0:2]


# speedup vs baseline: 8.3168x; 1.4639x over previous
import functools

import jax
import jax.numpy as jnp
from jax import lax
from jax.experimental import pallas as pl
from jax.experimental.pallas import tpu as pltpu
from jax.experimental.pallas import tpu_sc as plsc

N = 10000
E = 320000
EMB = 128
NC = 2
NS = 16
NW = NC * NS
ECH = E // NS
CAP = ECH + 224
BK = 64
NB = CAP // BK
NPAD = 10240
RPT = NPAD // NS
TRASH = N
GK = 4
FP32 = jnp.float32


def _leaky(x):
    return jnp.where(x >= 0, x, 0.01 * x)


def _dot(a, b):
    return jnp.dot(a, b, preferred_element_type=FP32,
                   precision=lax.Precision.HIGHEST)


def _part_body(src_hbm, dst_hbm, et_hbm, psrc_hbm, pdst_hbm, pcnt_hbm,
               cnt_hbm, srcv, dstv, etv, psb, pdb, cntv, cntl):
    c = lax.axis_index("c")
    s = lax.axis_index("s")
    wid = s * NC + c
    base = s * ECH
    pltpu.sync_copy(src_hbm.at[pl.ds(base, ECH)], srcv)
    pltpu.sync_copy(dst_hbm.at[pl.ds(base, ECH)], dstv)
    pltpu.sync_copy(et_hbm.at[pl.ds(base, ECH)], etv)

    zeros16f = jnp.zeros((16,), FP32)

    def zbody(i, carry):
        cntl[pl.ds(i * 16, 16)] = zeros16f
        return carry

    lax.fori_loop(0, NPAD // 16, zbody, jnp.int32(0))

    ones16f = jnp.ones((16,), FP32)

    def body(i, n):
        sl = pl.ds(i * 16, 16)
        m = etv[sl] == c
        mv = m.astype(jnp.int32)
        pos = n + plsc.cumsum(mv) - 1
        plsc.store_scatter(psb, [pos], srcv[sl], mask=m)
        plsc.store_scatter(pdb, [pos], dstv[sl], mask=m)
        plsc.addupdate_scatter(cntl, [dstv[sl]], ones16f, mask=m)
        return n + jnp.sum(mv)

    n = lax.fori_loop(0, ECH // 16, body, jnp.int32(0))
    pltpu.sync_copy(cntl, cnt_hbm.at[wid])

    zeros16 = jnp.zeros((16,), jnp.int32)
    trash16 = jnp.full((16,), TRASH, jnp.int32)
    for t in range(8):
        psb[pl.ds(n + t * 16, 16)] = zeros16
        pdb[pl.ds(n + t * 16, 16)] = trash16

    cntv[...] = jnp.full((16,), n, jnp.int32)
    pltpu.sync_copy(psb, psrc_hbm.at[wid])
    pltpu.sync_copy(pdb, pdst_hbm.at[wid])
    pltpu.sync_copy(cntv, pcnt_hbm.at[wid])


_partition = pl.kernel(
    _part_body,
    out_type=[
        jax.ShapeDtypeStruct((NW, CAP), jnp.int32),
        jax.ShapeDtypeStruct((NW, CAP), jnp.int32),
        jax.ShapeDtypeStruct((NW, 16), jnp.int32),
        jax.ShapeDtypeStruct((NW, NPAD), FP32),
    ],
    mesh=plsc.VectorSubcoreMesh(core_axis_name="c", subcore_axis_name="s",
                                num_cores=NC, num_subcores=NS),
    scratch_types=[
        pltpu.VMEM((ECH,), jnp.int32),
        pltpu.VMEM((ECH,), jnp.int32),
        pltpu.VMEM((ECH,), jnp.int32),
        pltpu.VMEM((CAP,), jnp.int32),
        pltpu.VMEM((CAP,), jnp.int32),
        pltpu.VMEM((16,), jnp.int32),
        pltpu.VMEM((NPAD,), FP32),
    ],
    compiler_params=pltpu.CompilerParams(needs_layout_passes=False),
)


NZ = RPT // BK
TAIL = RPT - NZ * BK


def _agg_body(x_hbm, psrc_hbm, pdst_hbm, pcnt_hbm, zrow_hbm, sums_hbm,
              psb, pdb, pcb, rowbuf, idxb, sem, acc_sh):
    c = lax.axis_index("c")
    s = lax.axis_index("s")
    wid = s * NC + c
    row0 = s * RPT

    pltpu.sync_copy(pcnt_hbm.at[wid], pcb)
    n = jnp.max(pcb[...])
    nb = (n + BK - 1) // BK

    pltpu.sync_copy(zrow_hbm.at[pl.ds(0, BK)], rowbuf.at[0])
    for b in range(NZ):
        for k in range(BK // 16):
            sl = pl.ds(k * 16, 16)
            idxb[sl] = lax.iota(jnp.int32, 16) + (row0 + b * BK + k * 16)
        pltpu.sync_copy(rowbuf.at[0], acc_sh.at[idxb])

    plsc.subcore_barrier()

    def group(g, carry):
        base = wid * NB + g * GK
        pltpu.sync_copy(psrc_hbm.at[pl.ds(base, GK)], psb)
        pltpu.sync_copy(pdst_hbm.at[pl.ds(base, GK)], pdb)
        for t in range(GK):
            pltpu.async_copy(x_hbm.at[psb.at[t]], rowbuf.at[t], sem)
        for t in range(GK):
            pltpu.make_async_copy(x_hbm.at[psb.at[t]], rowbuf.at[t],
                                  sem).wait()
        for t in range(GK):
            pltpu.sync_copy(rowbuf.at[t], acc_sh.at[pdb.at[t]], add=True)
        return carry

    ngf = nb // GK
    lax.fori_loop(0, ngf, group, jnp.int32(0))

    def step(j, carry):
        base = wid * NB + j
        pltpu.sync_copy(psrc_hbm.at[pl.ds(base, 1)], psb.at[pl.ds(0, 1)])
        pltpu.sync_copy(pdst_hbm.at[pl.ds(base, 1)], pdb.at[pl.ds(0, 1)])
        pltpu.async_copy(x_hbm.at[psb.at[0]], rowbuf.at[0], sem).wait()
        pltpu.sync_copy(rowbuf.at[0], acc_sh.at[pdb.at[0]], add=True)
        return carry

    lax.fori_loop(ngf * GK, nb, step, jnp.int32(0))

    plsc.subcore_barrier()

    out0 = c * NPAD + row0
    for b in range(NZ):
        for k in range(BK // 16):
            sl = pl.ds(k * 16, 16)
            idxb[sl] = lax.iota(jnp.int32, 16) + (row0 + b * BK + k * 16)
        pltpu.async_copy(acc_sh.at[idxb], rowbuf.at[0], sem).wait()
        pltpu.sync_copy(rowbuf.at[0], sums_hbm.at[pl.ds(out0 + b * BK, BK)])


_aggregate = pl.kernel(
    _agg_body,
    out_type=jax.ShapeDtypeStruct((NC * NPAD, 128), FP32),
    mesh=plsc.VectorSubcoreMesh(core_axis_name="c", subcore_axis_name="s",
                                num_cores=NC, num_subcores=NS),
    scratch_types=[
        pltpu.VMEM((GK, BK), jnp.int32),
        pltpu.VMEM((GK, BK), jnp.int32),
        pltpu.VMEM((16,), jnp.int32),
        pltpu.VMEM((GK, BK, 128), FP32),
        pltpu.VMEM((BK,), jnp.int32),
        pltpu.SemaphoreType.DMA,
        pltpu.VMEM_SHARED((NPAD, 128), FP32),
    ],
    compiler_params=pltpu.CompilerParams(needs_layout_passes=False),
)


R = 400
GRID = N // R


def _t0_body(des_ref, tw_ref, sm_ref, wd_ref, wt_ref, ws_ref, bf_ref,
             win_ref, bin_ref, out_ref):
    h = (_dot(des_ref[...], wd_ref[...]) + _dot(tw_ref[...], wt_ref[...])
         + _dot(sm_ref[...], ws_ref[...]) + bf_ref[...])
    h = _leaky(h)
    out_ref[...] = _leaky(_dot(h, win_ref[...]) + bin_ref[...])


_t0 = pl.pallas_call(
    _t0_body,
    out_shape=jax.ShapeDtypeStruct((N, 128), FP32),
    grid=(GRID,),
    in_specs=[
        pl.BlockSpec((R, 768), lambda i: (i, 0)),
        pl.BlockSpec((R, 768), lambda i: (i, 0)),
        pl.BlockSpec((R, 24), lambda i: (i, 0)),
        pl.BlockSpec((768, 128), lambda i: (0, 0)),
        pl.BlockSpec((768, 128), lambda i: (0, 0)),
        pl.BlockSpec((24, 128), lambda i: (0, 0)),
        pl.BlockSpec((1, 128), lambda i: (0, 0)),
        pl.BlockSpec((128, 128), lambda i: (0, 0)),
        pl.BlockSpec((1, 128), lambda i: (0, 0)),
    ],
    out_specs=pl.BlockSpec((R, 128), lambda i: (i, 0)),
)


def _combine(s0, s1, c0, c1, x, w0, w1, wr, b):
    inv0 = 1.0 / jnp.maximum(jnp.sum(c0, axis=1, keepdims=True), 1.0)
    inv1 = 1.0 / jnp.maximum(jnp.sum(c1, axis=1, keepdims=True), 1.0)
    return (_dot(s0 * inv0, w0) + _dot(s1 * inv1, w1) + _dot(x, wr) + b)


def _t1_body(s0_ref, s1_ref, c0_ref, c1_ref, x_ref, w0_ref, w1_ref, wr_ref,
             b_ref, out_ref):
    out_ref[...] = _combine(s0_ref[0], s1_ref[0], c0_ref[0], c1_ref[0],
                            x_ref[...], w0_ref[...], w1_ref[...], wr_ref[...],
                            b_ref[...])


def _t1h_body(s0_ref, s1_ref, c0_ref, c1_ref, x_ref, w0_ref, w1_ref, wr_ref,
              b_ref, wo1_ref, bo1_ref, wo2_ref, bo2_ref, out_ref):
    y = _combine(s0_ref[0], s1_ref[0], c0_ref[0], c1_ref[0], x_ref[...],
                 w0_ref[...], w1_ref[...], wr_ref[...], b_ref[...])
    z = _leaky(_dot(y, wo1_ref[...]) + bo1_ref[...])
    out_ref[...] = _dot(z, wo2_ref[...]) + bo2_ref[...]


_COMBINE_SPECS = [
    pl.BlockSpec((1, R, 128), lambda i: (0, i, 0)),
    pl.BlockSpec((1, R, 128), lambda i: (1, i, 0)),
    pl.BlockSpec((1, R, 16), lambda i: (0, i, 0)),
    pl.BlockSpec((1, R, 16), lambda i: (1, i, 0)),
    pl.BlockSpec((R, 128), lambda i: (i, 0)),
    pl.BlockSpec((128, 128), lambda i: (0, 0)),
    pl.BlockSpec((128, 128), lambda i: (0, 0)),
    pl.BlockSpec((128, 128), lambda i: (0, 0)),
    pl.BlockSpec((1, 128), lambda i: (0, 0)),
]

_t1 = pl.pallas_call(
    _t1_body,
    out_shape=jax.ShapeDtypeStruct((N, 128), FP32),
    grid=(GRID,),
    in_specs=_COMBINE_SPECS,
    out_specs=pl.BlockSpec((R, 128), lambda i: (i, 0)),
)

_t1h = pl.pallas_call(
    _t1h_body,
    out_shape=jax.ShapeDtypeStruct((N, 128), FP32),
    grid=(GRID,),
    in_specs=_COMBINE_SPECS + [
        pl.BlockSpec((128, 128), lambda i: (0, 0)),
        pl.BlockSpec((1, 128), lambda i: (0, 0)),
        pl.BlockSpec((128, 128), lambda i: (0, 0)),
        pl.BlockSpec((1, 128), lambda i: (0, 0)),
    ],
    out_specs=pl.BlockSpec((R, 128), lambda i: (i, 0)),
)


def kernel(des, tweet, num_prop, cat_prop, new_feature, edge_index, edge_type,
           W_des, b_des, W_tweet, b_tweet, W_num, b_num, W_cat, b_cat, W_new,
           b_new, W_in, b_in, rgcn1_w, rgcn1_root, rgcn1_b, rgcn2_w,
           rgcn2_root, rgcn2_b, W_out1, b_out1, W_out2, b_out2):
    src = edge_index[0]
    dst = edge_index[1]

    small = jnp.concatenate(
        [num_prop, cat_prop, new_feature, jnp.zeros((N, 5), FP32)], axis=1)
    wd = jnp.zeros((768, 128), FP32).at[:, 0:25].set(W_des)
    wt = jnp.zeros((768, 128), FP32).at[:, 25:53].set(W_tweet)
    ws = (jnp.zeros((24, 128), FP32)
          .at[0:7, 53:78].set(W_num)
          .at[7:18, 78:103].set(W_cat)
          .at[18:19, 103:128].set(W_new))
    bf = jnp.concatenate([b_des, b_tweet, b_num, b_cat, b_new]).reshape(1, 128)

    x0 = _t0(des, tweet, small, wd, wt, ws, bf, W_in, b_in.reshape(1, 128))

    psrc, pdst, pcnt, cnt = _partition(src, dst, edge_type)
    psrc3 = psrc.reshape(NW * NB, BK)
    pdst3 = pdst.reshape(NW * NB, BK)
    cnts = cnt.reshape(NS, NC, NPAD).transpose(1, 2, 0)

    zrow = jnp.zeros((RPT, 128), FP32)

    sums1 = _aggregate(x0, psrc3, pdst3, pcnt, zrow).reshape(NC, NPAD, 128)

    x1 = _t1(sums1, sums1, cnts, cnts, x0, rgcn1_w[0], rgcn1_w[1],
             rgcn1_root, rgcn1_b.reshape(1, 128))

    sums2 = _aggregate(x1, psrc3, pdst3, pcnt, zrow).reshape(NC, NPAD, 128)

    wo2 = jnp.zeros((128, 128), FP32).at[:, 0:2].set(W_out2)
    bo2 = jnp.zeros((1, 128), FP32).at[0, 0:2].set(b_out2)

    out = _t1h(sums2, sums2, cnts, cnts, x1, rgcn2_w[0], rgcn2_w[1],
               rgcn2_root, rgcn2_b.reshape(1, 128), W_out1,
               b_out1.reshape(1, 128), wo2, bo2)
    return out[:, 0:2]


# speedup vs baseline: 8.6627x; 1.0416x over previous
import functools

import jax
import jax.numpy as jnp
from jax import lax
from jax.experimental import pallas as pl
from jax.experimental.pallas import tpu as pltpu
from jax.experimental.pallas import tpu_sc as plsc

N = 10000
E = 320000
EMB = 128
NC = 2
NS = 16
NW = NC * NS
ECH = E // NS
CAP = ECH + 224
BK = 64
NB = CAP // BK
NPAD = 10240
RPT = NPAD // NS
TRASH = N
GK = 4
FP32 = jnp.float32


def _leaky(x):
    return jnp.where(x >= 0, x, 0.01 * x)


def _dot(a, b):
    return jnp.dot(a, b, preferred_element_type=FP32,
                   precision=lax.Precision.HIGHEST)


def _part_body(src_hbm, dst_hbm, et_hbm, psrc_hbm, pdst_hbm, pcnt_hbm,
               cnt_hbm, srcv, dstv, etv, psb, pdb, cntv, cntl):
    c = lax.axis_index("c")
    s = lax.axis_index("s")
    wid = s * NC + c
    base = s * ECH
    pltpu.sync_copy(src_hbm.at[pl.ds(base, ECH)], srcv)
    pltpu.sync_copy(dst_hbm.at[pl.ds(base, ECH)], dstv)
    pltpu.sync_copy(et_hbm.at[pl.ds(base, ECH)], etv)

    zeros16f = jnp.zeros((16,), FP32)

    def zbody(i, carry):
        cntl[pl.ds(i * 16, 16)] = zeros16f
        return carry

    lax.fori_loop(0, NPAD // 16, zbody, jnp.int32(0))

    ones16f = jnp.ones((16,), FP32)

    def body(i, n):
        sl = pl.ds(i * 16, 16)
        m = etv[sl] == c
        mv = m.astype(jnp.int32)
        pos = n + plsc.cumsum(mv) - 1
        plsc.store_scatter(psb, [pos], srcv[sl], mask=m)
        plsc.store_scatter(pdb, [pos], dstv[sl], mask=m)
        plsc.addupdate_scatter(cntl, [dstv[sl]], ones16f, mask=m)
        return n + jnp.sum(mv)

    n = lax.fori_loop(0, ECH // 16, body, jnp.int32(0))
    pltpu.sync_copy(cntl, cnt_hbm.at[wid])

    zeros16 = jnp.zeros((16,), jnp.int32)
    trash16 = jnp.full((16,), TRASH, jnp.int32)
    for t in range(8):
        psb[pl.ds(n + t * 16, 16)] = zeros16
        pdb[pl.ds(n + t * 16, 16)] = trash16

    cntv[...] = jnp.full((16,), n, jnp.int32)
    pltpu.sync_copy(psb, psrc_hbm.at[wid])
    pltpu.sync_copy(pdb, pdst_hbm.at[wid])
    pltpu.sync_copy(cntv, pcnt_hbm.at[wid])


_partition = pl.kernel(
    _part_body,
    out_type=[
        jax.ShapeDtypeStruct((NW, CAP), jnp.int32),
        jax.ShapeDtypeStruct((NW, CAP), jnp.int32),
        jax.ShapeDtypeStruct((NW, 16), jnp.int32),
        jax.ShapeDtypeStruct((NW, NPAD), FP32),
    ],
    mesh=plsc.VectorSubcoreMesh(core_axis_name="c", subcore_axis_name="s",
                                num_cores=NC, num_subcores=NS),
    scratch_types=[
        pltpu.VMEM((ECH,), jnp.int32),
        pltpu.VMEM((ECH,), jnp.int32),
        pltpu.VMEM((ECH,), jnp.int32),
        pltpu.VMEM((CAP,), jnp.int32),
        pltpu.VMEM((CAP,), jnp.int32),
        pltpu.VMEM((16,), jnp.int32),
        pltpu.VMEM((NPAD,), FP32),
    ],
    compiler_params=pltpu.CompilerParams(needs_layout_passes=False),
)


NZ = RPT // BK
TAIL = RPT - NZ * BK


def _agg_body(x_hbm, pidx_hbm, pcnt_hbm, zrow_hbm, sums_hbm,
              psb2, pcb, rowbuf, idxb, sem, acc_sh):
    c = lax.axis_index("c")
    s = lax.axis_index("s")
    wid = s * NC + c
    row0 = s * RPT

    pltpu.sync_copy(pcnt_hbm.at[wid], pcb)
    n = jnp.max(pcb[...])
    nb = (n + BK - 1) // BK

    pltpu.sync_copy(zrow_hbm.at[pl.ds(0, BK)], rowbuf.at[0])
    for b in range(NZ):
        for k in range(BK // 16):
            sl = pl.ds(k * 16, 16)
            idxb[sl] = lax.iota(jnp.int32, 16) + (row0 + b * BK + k * 16)
        pltpu.sync_copy(rowbuf.at[0], acc_sh.at[idxb])

    plsc.subcore_barrier()

    def group(g, carry):
        base = wid * NB + g * GK
        pltpu.sync_copy(pidx_hbm.at[pl.ds(base, GK)], psb2)
        for t in range(GK):
            pltpu.async_copy(x_hbm.at[psb2.at[t, 0]], rowbuf.at[t], sem)
        for t in range(GK):
            pltpu.make_async_copy(x_hbm.at[psb2.at[t, 0]], rowbuf.at[t],
                                  sem).wait()
        for t in range(GK):
            pltpu.sync_copy(rowbuf.at[t], acc_sh.at[psb2.at[t, 1]], add=True)
        return carry

    ngf = nb // GK
    lax.fori_loop(0, ngf, group, jnp.int32(0))

    def step(j, carry):
        base = wid * NB + j
        pltpu.sync_copy(pidx_hbm.at[pl.ds(base, 1)], psb2.at[pl.ds(0, 1)])
        pltpu.async_copy(x_hbm.at[psb2.at[0, 0]], rowbuf.at[0], sem).wait()
        pltpu.sync_copy(rowbuf.at[0], acc_sh.at[psb2.at[0, 1]], add=True)
        return carry

    lax.fori_loop(ngf * GK, nb, step, jnp.int32(0))

    plsc.subcore_barrier()

    out0 = c * NPAD + row0
    for b in range(NZ):
        for k in range(BK // 16):
            sl = pl.ds(k * 16, 16)
            idxb[sl] = lax.iota(jnp.int32, 16) + (row0 + b * BK + k * 16)
        pltpu.async_copy(acc_sh.at[idxb], rowbuf.at[0], sem).wait()
        pltpu.sync_copy(rowbuf.at[0], sums_hbm.at[pl.ds(out0 + b * BK, BK)])


_aggregate = pl.kernel(
    _agg_body,
    out_type=jax.ShapeDtypeStruct((NC * NPAD, 128), FP32),
    mesh=plsc.VectorSubcoreMesh(core_axis_name="c", subcore_axis_name="s",
                                num_cores=NC, num_subcores=NS),
    scratch_types=[
        pltpu.VMEM((GK, 2, BK), jnp.int32),
        pltpu.VMEM((16,), jnp.int32),
        pltpu.VMEM((GK, BK, 128), FP32),
        pltpu.VMEM((BK,), jnp.int32),
        pltpu.SemaphoreType.DMA,
        pltpu.VMEM_SHARED((NPAD, 128), FP32),
    ],
    compiler_params=pltpu.CompilerParams(needs_layout_passes=False),
)


R = 400
GRID = N // R


def _t0_body(des_ref, tw_ref, sm_ref, wd_ref, wt_ref, ws_ref, bf_ref,
             win_ref, bin_ref, out_ref):
    h = (_dot(des_ref[...], wd_ref[...]) + _dot(tw_ref[...], wt_ref[...])
         + _dot(sm_ref[...], ws_ref[...]) + bf_ref[...])
    h = _leaky(h)
    out_ref[...] = _leaky(_dot(h, win_ref[...]) + bin_ref[...])


_t0 = pl.pallas_call(
    _t0_body,
    out_shape=jax.ShapeDtypeStruct((N, 128), FP32),
    grid=(GRID,),
    in_specs=[
        pl.BlockSpec((R, 768), lambda i: (i, 0)),
        pl.BlockSpec((R, 768), lambda i: (i, 0)),
        pl.BlockSpec((R, 24), lambda i: (i, 0)),
        pl.BlockSpec((768, 128), lambda i: (0, 0)),
        pl.BlockSpec((768, 128), lambda i: (0, 0)),
        pl.BlockSpec((24, 128), lambda i: (0, 0)),
        pl.BlockSpec((1, 128), lambda i: (0, 0)),
        pl.BlockSpec((128, 128), lambda i: (0, 0)),
        pl.BlockSpec((1, 128), lambda i: (0, 0)),
    ],
    out_specs=pl.BlockSpec((R, 128), lambda i: (i, 0)),
)


def _combine(s0, s1, c0, c1, x, w0, w1, wr, b):
    inv0 = 1.0 / jnp.maximum(jnp.sum(c0, axis=1, keepdims=True), 1.0)
    inv1 = 1.0 / jnp.maximum(jnp.sum(c1, axis=1, keepdims=True), 1.0)
    return (_dot(s0 * inv0, w0) + _dot(s1 * inv1, w1) + _dot(x, wr) + b)


def _t1_body(s0_ref, s1_ref, c0_ref, c1_ref, x_ref, w0_ref, w1_ref, wr_ref,
             b_ref, out_ref):
    out_ref[...] = _combine(s0_ref[0], s1_ref[0], c0_ref[0], c1_ref[0],
                            x_ref[...], w0_ref[...], w1_ref[...], wr_ref[...],
                            b_ref[...])


def _t1h_body(s0_ref, s1_ref, c0_ref, c1_ref, x_ref, w0_ref, w1_ref, wr_ref,
              b_ref, wo1_ref, bo1_ref, wo2_ref, bo2_ref, out_ref):
    y = _combine(s0_ref[0], s1_ref[0], c0_ref[0], c1_ref[0], x_ref[...],
                 w0_ref[...], w1_ref[...], wr_ref[...], b_ref[...])
    z = _leaky(_dot(y, wo1_ref[...]) + bo1_ref[...])
    out_ref[...] = _dot(z, wo2_ref[...]) + bo2_ref[...]


_COMBINE_SPECS = [
    pl.BlockSpec((1, R, 128), lambda i: (0, i, 0)),
    pl.BlockSpec((1, R, 128), lambda i: (1, i, 0)),
    pl.BlockSpec((1, R, 16), lambda i: (0, i, 0)),
    pl.BlockSpec((1, R, 16), lambda i: (1, i, 0)),
    pl.BlockSpec((R, 128), lambda i: (i, 0)),
    pl.BlockSpec((128, 128), lambda i: (0, 0)),
    pl.BlockSpec((128, 128), lambda i: (0, 0)),
    pl.BlockSpec((128, 128), lambda i: (0, 0)),
    pl.BlockSpec((1, 128), lambda i: (0, 0)),
]

_t1 = pl.pallas_call(
    _t1_body,
    out_shape=jax.ShapeDtypeStruct((N, 128), FP32),
    grid=(GRID,),
    in_specs=_COMBINE_SPECS,
    out_specs=pl.BlockSpec((R, 128), lambda i: (i, 0)),
)

_t1h = pl.pallas_call(
    _t1h_body,
    out_shape=jax.ShapeDtypeStruct((N, 128), FP32),
    grid=(GRID,),
    in_specs=_COMBINE_SPECS + [
        pl.BlockSpec((128, 128), lambda i: (0, 0)),
        pl.BlockSpec((1, 128), lambda i: (0, 0)),
        pl.BlockSpec((128, 128), lambda i: (0, 0)),
        pl.BlockSpec((1, 128), lambda i: (0, 0)),
    ],
    out_specs=pl.BlockSpec((R, 128), lambda i: (i, 0)),
)


def kernel(des, tweet, num_prop, cat_prop, new_feature, edge_index, edge_type,
           W_des, b_des, W_tweet, b_tweet, W_num, b_num, W_cat, b_cat, W_new,
           b_new, W_in, b_in, rgcn1_w, rgcn1_root, rgcn1_b, rgcn2_w,
           rgcn2_root, rgcn2_b, W_out1, b_out1, W_out2, b_out2):
    src = edge_index[0]
    dst = edge_index[1]

    small = jnp.concatenate(
        [num_prop, cat_prop, new_feature, jnp.zeros((N, 5), FP32)], axis=1)
    wd = jnp.zeros((768, 128), FP32).at[:, 0:25].set(W_des)
    wt = jnp.zeros((768, 128), FP32).at[:, 25:53].set(W_tweet)
    ws = (jnp.zeros((24, 128), FP32)
          .at[0:7, 53:78].set(W_num)
          .at[7:18, 78:103].set(W_cat)
          .at[18:19, 103:128].set(W_new))
    bf = jnp.concatenate([b_des, b_tweet, b_num, b_cat, b_new]).reshape(1, 128)

    x0 = _t0(des, tweet, small, wd, wt, ws, bf, W_in, b_in.reshape(1, 128))

    psrc, pdst, pcnt, cnt = _partition(src, dst, edge_type)
    pidx = jnp.concatenate([psrc.reshape(NW * NB, 1, BK),
                            pdst.reshape(NW * NB, 1, BK)], axis=1)
    cnts = cnt.reshape(NS, NC, NPAD).transpose(1, 2, 0)

    zrow = jnp.zeros((RPT, 128), FP32)

    sums1 = _aggregate(x0, pidx, pcnt, zrow).reshape(NC, NPAD, 128)

    x1 = _t1(sums1, sums1, cnts, cnts, x0, rgcn1_w[0], rgcn1_w[1],
             rgcn1_root, rgcn1_b.reshape(1, 128))

    sums2 = _aggregate(x1, pidx, pcnt, zrow).reshape(NC, NPAD, 128)

    wo2 = jnp.zeros((128, 128), FP32).at[:, 0:2].set(W_out2)
    bo2 = jnp.zeros((1, 128), FP32).at[0, 0:2].set(b_out2)

    out = _t1h(sums2, sums2, cnts, cnts, x1, rgcn2_w[0], rgcn2_w[1],
               rgcn2_root, rgcn2_b.reshape(1, 128), W_out1,
               b_out1.reshape(1, 128), wo2, bo2)
    return out[:, 0:2]
